# baseline (speedup 1.0000x reference)
import functools

import jax
import jax.numpy as jnp
from jax import lax
from jax.experimental import pallas as pl
from jax.experimental.pallas import tpu as pltpu
from jax.experimental.pallas import tpu_sc as plsc

VOCAB = 5
EMBED_DIM = 256
MAX_LEN = 512
EPS = 1e-5

BATCH_BLK = 32


def _combine_kernel(tok_ref, pos_ref, gamma_ref, beta_ref, out_ref):
    emb = tok_ref[...][:, None, :] + pos_ref[...][None, :, :]
    mean = jnp.mean(emb, axis=-1, keepdims=True)
    var = jnp.mean(jnp.square(emb - mean), axis=-1, keepdims=True)
    normed = (emb - mean) * jax.lax.rsqrt(var + EPS)
    out_ref[...] = normed * gamma_ref[...][None, None, :] + beta_ref[...][None, None, :]


def _expand_kernel(x_ref, comb_ref, out_ref):
    xb = x_ref[...]
    c = comb_ref[...]
    sel = xb[:, :, None]
    r = jnp.where(sel == 0, c[0][None], c[4][None])
    r = jnp.where(sel == 1, c[1][None], r)
    r = jnp.where(sel == 2, c[2][None], r)
    r = jnp.where(sel == 3, c[3][None], r)
    out_ref[...] = r


def _idx_kernel(x_ref, idx_ref):
    s_iota = lax.broadcasted_iota(jnp.int32, x_ref.shape, 1)
    idx_ref[...] = x_ref[...] * MAX_LEN + s_iota


def _combine_win_kernel(tok_ref, pos_ref, gamma_ref, beta_ref, out_ref):
    emb = tok_ref[...][:, None, :] + pos_ref[...][None, :, :]
    mean = jnp.mean(emb, axis=-1, keepdims=True)
    var = jnp.mean(jnp.square(emb - mean), axis=-1, keepdims=True)
    normed = (emb - mean) * jax.lax.rsqrt(var + EPS)
    normed = normed * gamma_ref[...][None, None, :] + beta_ref[...][None, None, :]
    v, s, d = normed.shape
    n_win = s // S_WIN
    out_ref[...] = normed.reshape(v, n_win, S_WIN, d).transpose(1, 0, 2, 3)


def _xt_kernel(x_ref, out_ref):
    out_ref[...] = x_ref[...].T


NUM_WORKERS = 32
SC_CHUNK = 64


NBUF = 4
LOOKAHEAD = 2


def _sc_expand(total_rows, dim):
    b_per_w = total_rows // NUM_WORKERS
    n_chunks = b_per_w // SC_CHUNK
    mesh = plsc.VectorSubcoreMesh(core_axis_name="c", subcore_axis_name="s")

    @functools.partial(
        pl.kernel,
        mesh=mesh,
        out_type=jax.ShapeDtypeStruct((total_rows, dim), jnp.float32),
        scratch_types=(
            [pltpu.VMEM((b_per_w,), jnp.int32)]
            + [pltpu.VMEM((SC_CHUNK, dim), jnp.float32)] * NBUF
            + [pltpu.SemaphoreType.DMA] * (2 * NBUF)
        ),
    )
    def expand(idx_hbm, table_hbm, out_hbm, idx_v, *bufs_sems):
        bufs = bufs_sems[:NBUF]
        gsems = bufs_sems[NBUF:2 * NBUF]
        ssems = bufs_sems[2 * NBUF:]
        wid = lax.axis_index("s") * 2 + lax.axis_index("c")
        base = wid * b_per_w
        pltpu.sync_copy(idx_hbm.at[pl.ds(base, b_per_w)], idx_v)

        def gather(c, b):
            return pltpu.make_async_copy(
                table_hbm.at[idx_v.at[pl.ds(c * SC_CHUNK, SC_CHUNK)]],
                bufs[b], gsems[b])

        def scatter(c, b):
            return pltpu.make_async_copy(
                bufs[b], out_hbm.at[pl.ds(base + c * SC_CHUNK, SC_CHUNK)],
                ssems[b])

        for p in range(LOOKAHEAD):
            gather(p, p % NBUF).start()

        def body(c2, _):
            for u in range(NBUF):
                c = c2 * NBUF + u
                b = u
                nxt = c + LOOKAHEAD
                nb = (u + LOOKAHEAD) % NBUF

                @pl.when(jnp.logical_and(nxt < n_chunks, c >= NBUF - LOOKAHEAD))
                def _():
                    scatter(nxt - NBUF, nb).wait()

                @pl.when(nxt < n_chunks)
                def _():
                    gather(nxt, nb).start()

                gather(c, b).wait()
                scatter(c, b).start()
            return 0

        lax.fori_loop(0, n_chunks // NBUF, body, 0)
        for u in range(NBUF):
            scatter(n_chunks - NBUF + u, u).wait()

    return expand


S_WIN = 16
D_UNROLL = 8


def _sc_expand_win(batch, seq, dim, vocab):
    tbl_words = vocab * S_WIN * dim
    x_words = S_WIN * batch
    mesh = plsc.VectorSubcoreMesh(core_axis_name="c", subcore_axis_name="s")

    @functools.partial(
        pl.kernel,
        mesh=mesh,
        out_type=jax.ShapeDtypeStruct((batch * seq, dim), jnp.float32),
        compiler_params=pltpu.CompilerParams(needs_layout_passes=False),
        scratch_types=[
            pltpu.VMEM_SHARED((32 * tbl_words,), jnp.float32),
            pltpu.VMEM((x_words,), jnp.int32),
            pltpu.VMEM((S_WIN, dim), jnp.float32),
            pltpu.VMEM((S_WIN, dim), jnp.float32),
            pltpu.SemaphoreType.DMA,
            pltpu.SemaphoreType.DMA,
            pltpu.SemaphoreType.DMA,
            pltpu.SemaphoreType.DMA,
        ],
    )
    def expand(xt_hbm, table_hbm, out_hbm, tbl_sp, x_v, buf0, buf1,
               lsem, s0, s1, fsem):
        sid = lax.axis_index("s")
        wid = sid * 2 + lax.axis_index("c")
        s_base = wid * S_WIN
        tb_off = wid * tbl_words

        pltpu.make_async_copy(
            table_hbm.at[pl.ds(sid * 2 * tbl_words, 2 * tbl_words)],
            tbl_sp.at[pl.ds(sid * 2 * tbl_words, 2 * tbl_words)], lsem).start()
        pltpu.make_async_copy(
            xt_hbm.at[pl.ds(wid * x_words, x_words)], x_v, lsem).start()
        pltpu.make_async_copy(
            table_hbm.at[pl.ds(sid * 2 * tbl_words, 2 * tbl_words)],
            tbl_sp.at[pl.ds(sid * 2 * tbl_words, 2 * tbl_words)], lsem).wait()
        pltpu.make_async_copy(
            xt_hbm.at[pl.ds(wid * x_words, x_words)], x_v, lsem).wait()
        plsc.subcore_barrier()

        iota = lax.iota(jnp.int32, 16)
        out_base = s_base + iota
        bufs = (buf0, buf1)
        ssems = (s0, s1)

        LAG = 8

        def rows_of(b_loc):
            xv = x_v[pl.ds(b_loc * S_WIN, S_WIN)]
            rb_vec = tb_off + (xv * S_WIN + iota) * dim
            out_row0 = b_loc * seq + s_base
            copies = []
            for l in range(S_WIN):
                rb = jnp.reshape(lax.slice(rb_vec, (l,), (l + 1,)), ())
                rb = pl.multiple_of(rb, dim)
                copies.append(pltpu.make_async_copy(
                    tbl_sp.at[pl.ds(rb, dim)],
                    out_hbm.at[out_row0 + l], fsem))
            return copies

        def fire(b_loc):
            for c in rows_of(b_loc):
                c.start()

        def drain_one_b():
            d = pltpu.make_async_copy(
                tbl_sp.at[pl.ds(0, dim)], out_hbm.at[0], fsem)
            for _ in range(S_WIN):
                d.wait()

        for p in range(LAG):
            fire(p)

        def body(b_loc, _):
            fire(b_loc + LAG)
            drain_one_b()
            return 0

        lax.fori_loop(0, batch - LAG, body, 0)
        for p in range(LAG):
            drain_one_b()

    return expand


@functools.partial(jax.jit, static_argnums=())
def kernel(x, token_table, pos_table, gamma, beta):
    batch, seq = x.shape
    vocab, dim = token_table.shape

    combined = pl.pallas_call(
        _combine_kernel,
        out_shape=jax.ShapeDtypeStruct((vocab, seq, dim), jnp.float32),
    )(token_table, pos_table[:seq], gamma, beta)

    x = x.astype(jnp.int32)
    combined_win = pl.pallas_call(
        _combine_win_kernel,
        out_shape=jax.ShapeDtypeStruct((seq // S_WIN, vocab, S_WIN, dim),
                                       jnp.float32),
    )(token_table, pos_table[:seq], gamma, beta)
    xw = jnp.transpose(x.reshape(batch, seq // S_WIN, S_WIN),
                       (1, 0, 2)).reshape(-1)
    out = _sc_expand_win(batch, seq, dim, vocab)(
        xw, combined_win.reshape(-1))
    return out.reshape(batch, seq, dim)


# speedup vs baseline: 3.3568x; 3.3568x over previous
import functools

import jax
import jax.numpy as jnp
from jax import lax
from jax.experimental import pallas as pl
from jax.experimental.pallas import tpu as pltpu
from jax.experimental.pallas import tpu_sc as plsc

VOCAB = 5
EMBED_DIM = 256
MAX_LEN = 512
EPS = 1e-5

BATCH_BLK = 32


def _combine_kernel(tok_ref, pos_ref, gamma_ref, beta_ref, out_ref):
    emb = tok_ref[...][:, None, :] + pos_ref[...][None, :, :]
    mean = jnp.mean(emb, axis=-1, keepdims=True)
    var = jnp.mean(jnp.square(emb - mean), axis=-1, keepdims=True)
    normed = (emb - mean) * jax.lax.rsqrt(var + EPS)
    out_ref[...] = normed * gamma_ref[...][None, None, :] + beta_ref[...][None, None, :]


def _expand_kernel(x_ref, comb_ref, out_ref):
    xb = x_ref[...]
    c = comb_ref[...]
    sel = xb[:, :, None]
    r = jnp.where(sel == 0, c[0][None], c[4][None])
    r = jnp.where(sel == 1, c[1][None], r)
    r = jnp.where(sel == 2, c[2][None], r)
    r = jnp.where(sel == 3, c[3][None], r)
    out_ref[...] = r


def _idx_kernel(x_ref, idx_ref):
    s_iota = lax.broadcasted_iota(jnp.int32, x_ref.shape, 1)
    idx_ref[...] = x_ref[...] * MAX_LEN + s_iota


def _combine_win_kernel(tok_ref, pos_ref, gamma_ref, beta_ref, out_ref):
    emb = tok_ref[...][:, None, :] + pos_ref[...][None, :, :]
    mean = jnp.mean(emb, axis=-1, keepdims=True)
    var = jnp.mean(jnp.square(emb - mean), axis=-1, keepdims=True)
    normed = (emb - mean) * jax.lax.rsqrt(var + EPS)
    normed = normed * gamma_ref[...][None, None, :] + beta_ref[...][None, None, :]
    v, s, d = normed.shape
    n_win = s // S_WIN
    out_ref[...] = normed.reshape(v, n_win, S_WIN, d).transpose(1, 0, 2, 3)


def _xt_kernel(x_ref, out_ref):
    out_ref[...] = x_ref[...].T


NUM_WORKERS = 32
SC_CHUNK = 64


NBUF = 4
LOOKAHEAD = 2


def _sc_expand(total_rows, dim):
    b_per_w = total_rows // NUM_WORKERS
    n_chunks = b_per_w // SC_CHUNK
    mesh = plsc.VectorSubcoreMesh(core_axis_name="c", subcore_axis_name="s")

    @functools.partial(
        pl.kernel,
        mesh=mesh,
        out_type=jax.ShapeDtypeStruct((total_rows, dim), jnp.float32),
        scratch_types=(
            [pltpu.VMEM((b_per_w,), jnp.int32)]
            + [pltpu.VMEM((SC_CHUNK, dim), jnp.float32)] * NBUF
            + [pltpu.SemaphoreType.DMA] * (2 * NBUF)
        ),
    )
    def expand(idx_hbm, table_hbm, out_hbm, idx_v, *bufs_sems):
        bufs = bufs_sems[:NBUF]
        gsems = bufs_sems[NBUF:2 * NBUF]
        ssems = bufs_sems[2 * NBUF:]
        wid = lax.axis_index("s") * 2 + lax.axis_index("c")
        base = wid * b_per_w
        pltpu.sync_copy(idx_hbm.at[pl.ds(base, b_per_w)], idx_v)

        def gather(c, b):
            return pltpu.make_async_copy(
                table_hbm.at[idx_v.at[pl.ds(c * SC_CHUNK, SC_CHUNK)]],
                bufs[b], gsems[b])

        def scatter(c, b):
            return pltpu.make_async_copy(
                bufs[b], out_hbm.at[pl.ds(base + c * SC_CHUNK, SC_CHUNK)],
                ssems[b])

        for p in range(LOOKAHEAD):
            gather(p, p % NBUF).start()

        def body(c2, _):
            for u in range(NBUF):
                c = c2 * NBUF + u
                b = u
                nxt = c + LOOKAHEAD
                nb = (u + LOOKAHEAD) % NBUF

                @pl.when(jnp.logical_and(nxt < n_chunks, c >= NBUF - LOOKAHEAD))
                def _():
                    scatter(nxt - NBUF, nb).wait()

                @pl.when(nxt < n_chunks)
                def _():
                    gather(nxt, nb).start()

                gather(c, b).wait()
                scatter(c, b).start()
            return 0

        lax.fori_loop(0, n_chunks // NBUF, body, 0)
        for u in range(NBUF):
            scatter(n_chunks - NBUF + u, u).wait()

    return expand


S_WIN = 16
D_UNROLL = 8


def _sc_expand_win(batch, seq, dim, vocab):
    tbl_words = vocab * S_WIN * dim
    x_words = S_WIN * batch
    mesh = plsc.VectorSubcoreMesh(core_axis_name="c", subcore_axis_name="s")

    @functools.partial(
        pl.kernel,
        mesh=mesh,
        out_type=jax.ShapeDtypeStruct((batch * seq, dim), jnp.float32),
        compiler_params=pltpu.CompilerParams(needs_layout_passes=False),
        scratch_types=[
            pltpu.VMEM_SHARED((32 * tbl_words,), jnp.float32),
            pltpu.VMEM((x_words,), jnp.int32),
            pltpu.VMEM((S_WIN, dim), jnp.float32),
            pltpu.VMEM((S_WIN, dim), jnp.float32),
            pltpu.SemaphoreType.DMA,
            pltpu.SemaphoreType.DMA,
            pltpu.SemaphoreType.DMA,
            pltpu.SemaphoreType.DMA,
        ],
    )
    def expand(xt_hbm, table_hbm, out_hbm, tbl_sp, x_v, buf0, buf1,
               lsem, s0, s1, fsem):
        sid = lax.axis_index("s")
        wid = sid * 2 + lax.axis_index("c")
        s_base = wid * S_WIN
        tb_off = wid * tbl_words

        pltpu.make_async_copy(
            table_hbm.at[pl.ds(sid * 2 * tbl_words, 2 * tbl_words)],
            tbl_sp.at[pl.ds(sid * 2 * tbl_words, 2 * tbl_words)], lsem).start()
        pltpu.make_async_copy(
            xt_hbm.at[pl.ds(wid * x_words, x_words)], x_v, lsem).start()
        pltpu.make_async_copy(
            table_hbm.at[pl.ds(sid * 2 * tbl_words, 2 * tbl_words)],
            tbl_sp.at[pl.ds(sid * 2 * tbl_words, 2 * tbl_words)], lsem).wait()
        pltpu.make_async_copy(
            xt_hbm.at[pl.ds(wid * x_words, x_words)], x_v, lsem).wait()
        plsc.subcore_barrier()

        iota = lax.iota(jnp.int32, 16)
        out_base = s_base + iota
        bufs = (buf0, buf1)
        ssems = (s0, s1)

        def scatter(b_loc, u):
            return pltpu.make_async_copy(
                bufs[u], out_hbm.at[pl.ds(b_loc * seq + s_base, S_WIN), :],
                ssems[u])

        def fill(b_loc, buf):
            xv = x_v[pl.ds(b_loc * S_WIN, S_WIN)]
            rb_vec = tb_off + (xv * S_WIN + iota) * dim
            copies = []
            for l in range(S_WIN):
                rb = jnp.reshape(lax.slice(rb_vec, (l,), (l + 1,)), ())
                rb = pl.multiple_of(rb, dim)
                c = pltpu.make_async_copy(
                    tbl_sp.at[pl.ds(rb, dim)], buf.at[l], fsem)
                c.start()
                copies.append(c)
            for c in copies:
                c.wait()

        for u in range(2):
            fill(u, bufs[u])
            scatter(u, u).start()

        def body(b2, _):
            for u in range(2):
                b_loc = b2 * 2 + u
                scatter(b_loc - 2, u).wait()
                fill(b_loc, bufs[u])
                scatter(b_loc, u).start()
            return 0

        lax.fori_loop(1, batch // 2, body, 0)
        scatter(batch - 2, 0).wait()
        scatter(batch - 1, 1).wait()

    return expand


@functools.partial(jax.jit, static_argnums=())
def kernel(x, token_table, pos_table, gamma, beta):
    batch, seq = x.shape
    vocab, dim = token_table.shape

    combined = pl.pallas_call(
        _combine_kernel,
        out_shape=jax.ShapeDtypeStruct((vocab, seq, dim), jnp.float32),
    )(token_table, pos_table[:seq], gamma, beta)

    x = x.astype(jnp.int32)
    combined_win = pl.pallas_call(
        _combine_win_kernel,
        out_shape=jax.ShapeDtypeStruct((seq // S_WIN, vocab, S_WIN, dim),
                                       jnp.float32),
    )(token_table, pos_table[:seq], gamma, beta)
    xw = jnp.transpose(x.reshape(batch, seq // S_WIN, S_WIN),
                       (1, 0, 2)).reshape(-1)
    out = _sc_expand_win(batch, seq, dim, vocab)(
        xw, combined_win.reshape(-1))
    return out.reshape(batch, seq, dim)


# speedup vs baseline: 3.3711x; 1.0043x over previous
import functools

import jax
import jax.numpy as jnp
from jax import lax
from jax.experimental import pallas as pl
from jax.experimental.pallas import tpu as pltpu
from jax.experimental.pallas import tpu_sc as plsc

VOCAB = 5
EMBED_DIM = 256
MAX_LEN = 512
EPS = 1e-5

BATCH_BLK = 32


def _combine_kernel(tok_ref, pos_ref, gamma_ref, beta_ref, out_ref):
    emb = tok_ref[...][:, None, :] + pos_ref[...][None, :, :]
    mean = jnp.mean(emb, axis=-1, keepdims=True)
    var = jnp.mean(jnp.square(emb - mean), axis=-1, keepdims=True)
    normed = (emb - mean) * jax.lax.rsqrt(var + EPS)
    out_ref[...] = normed * gamma_ref[...][None, None, :] + beta_ref[...][None, None, :]


def _expand_kernel(x_ref, comb_ref, out_ref):
    xb = x_ref[...]
    c = comb_ref[...]
    sel = xb[:, :, None]
    r = jnp.where(sel == 0, c[0][None], c[4][None])
    r = jnp.where(sel == 1, c[1][None], r)
    r = jnp.where(sel == 2, c[2][None], r)
    r = jnp.where(sel == 3, c[3][None], r)
    out_ref[...] = r


def _idx_kernel(x_ref, idx_ref):
    s_iota = lax.broadcasted_iota(jnp.int32, x_ref.shape, 1)
    idx_ref[...] = x_ref[...] * MAX_LEN + s_iota


def _combine_win_kernel(tok_ref, pos_ref, gamma_ref, beta_ref, out_ref):
    emb = tok_ref[...][:, None, :] + pos_ref[...][None, :, :]
    mean = jnp.mean(emb, axis=-1, keepdims=True)
    var = jnp.mean(jnp.square(emb - mean), axis=-1, keepdims=True)
    normed = (emb - mean) * jax.lax.rsqrt(var + EPS)
    normed = normed * gamma_ref[...][None, None, :] + beta_ref[...][None, None, :]
    v, s, d = normed.shape
    n_win = s // S_WIN
    out_ref[...] = normed.reshape(v, n_win, S_WIN, d).transpose(1, 0, 2, 3)


def _xt_kernel(x_ref, out_ref):
    out_ref[...] = x_ref[...].T


NUM_WORKERS = 32
SC_CHUNK = 64


NBUF = 4
LOOKAHEAD = 2


def _sc_expand(total_rows, dim):
    b_per_w = total_rows // NUM_WORKERS
    n_chunks = b_per_w // SC_CHUNK
    mesh = plsc.VectorSubcoreMesh(core_axis_name="c", subcore_axis_name="s")

    @functools.partial(
        pl.kernel,
        mesh=mesh,
        out_type=jax.ShapeDtypeStruct((total_rows, dim), jnp.float32),
        scratch_types=(
            [pltpu.VMEM((b_per_w,), jnp.int32)]
            + [pltpu.VMEM((SC_CHUNK, dim), jnp.float32)] * NBUF
            + [pltpu.SemaphoreType.DMA] * (2 * NBUF)
        ),
    )
    def expand(idx_hbm, table_hbm, out_hbm, idx_v, *bufs_sems):
        bufs = bufs_sems[:NBUF]
        gsems = bufs_sems[NBUF:2 * NBUF]
        ssems = bufs_sems[2 * NBUF:]
        wid = lax.axis_index("s") * 2 + lax.axis_index("c")
        base = wid * b_per_w
        pltpu.sync_copy(idx_hbm.at[pl.ds(base, b_per_w)], idx_v)

        def gather(c, b):
            return pltpu.make_async_copy(
                table_hbm.at[idx_v.at[pl.ds(c * SC_CHUNK, SC_CHUNK)]],
                bufs[b], gsems[b])

        def scatter(c, b):
            return pltpu.make_async_copy(
                bufs[b], out_hbm.at[pl.ds(base + c * SC_CHUNK, SC_CHUNK)],
                ssems[b])

        for p in range(LOOKAHEAD):
            gather(p, p % NBUF).start()

        def body(c2, _):
            for u in range(NBUF):
                c = c2 * NBUF + u
                b = u
                nxt = c + LOOKAHEAD
                nb = (u + LOOKAHEAD) % NBUF

                @pl.when(jnp.logical_and(nxt < n_chunks, c >= NBUF - LOOKAHEAD))
                def _():
                    scatter(nxt - NBUF, nb).wait()

                @pl.when(nxt < n_chunks)
                def _():
                    gather(nxt, nb).start()

                gather(c, b).wait()
                scatter(c, b).start()
            return 0

        lax.fori_loop(0, n_chunks // NBUF, body, 0)
        for u in range(NBUF):
            scatter(n_chunks - NBUF + u, u).wait()

    return expand


S_WIN = 16
D_UNROLL = 8


def _sc_expand_win(batch, seq, dim, vocab):
    tbl_words = vocab * S_WIN * dim
    x_words = S_WIN * batch
    mesh = plsc.VectorSubcoreMesh(core_axis_name="c", subcore_axis_name="s")

    @functools.partial(
        pl.kernel,
        mesh=mesh,
        out_type=jax.ShapeDtypeStruct((batch * seq, dim), jnp.float32),
        compiler_params=pltpu.CompilerParams(needs_layout_passes=False),
        scratch_types=[
            pltpu.VMEM_SHARED((32 * tbl_words,), jnp.float32),
            pltpu.VMEM((x_words,), jnp.int32),
            pltpu.VMEM((S_WIN, dim), jnp.float32),
            pltpu.VMEM((S_WIN, dim), jnp.float32),
            pltpu.SemaphoreType.DMA,
            pltpu.SemaphoreType.DMA,
            pltpu.SemaphoreType.DMA,
            pltpu.SemaphoreType.DMA,
        ],
    )
    def expand(xt_hbm, table_hbm, out_hbm, tbl_sp, x_v, buf0, buf1,
               lsem, s0, s1, fsem):
        sid = lax.axis_index("s")
        wid = sid * 2 + lax.axis_index("c")
        s_base = wid * S_WIN
        tb_off = wid * tbl_words

        pltpu.make_async_copy(
            table_hbm.at[pl.ds(sid * 2 * tbl_words, 2 * tbl_words)],
            tbl_sp.at[pl.ds(sid * 2 * tbl_words, 2 * tbl_words)], lsem).start()
        pltpu.make_async_copy(
            xt_hbm.at[pl.ds(wid * x_words, x_words)], x_v, lsem).start()
        pltpu.make_async_copy(
            table_hbm.at[pl.ds(sid * 2 * tbl_words, 2 * tbl_words)],
            tbl_sp.at[pl.ds(sid * 2 * tbl_words, 2 * tbl_words)], lsem).wait()
        pltpu.make_async_copy(
            xt_hbm.at[pl.ds(wid * x_words, x_words)], x_v, lsem).wait()
        plsc.subcore_barrier()

        iota = lax.iota(jnp.int32, 16)
        out_base = s_base + iota
        bufs = (buf0, buf1)
        ssems = (s0, s1)

        def scatter(b_loc, u):
            return pltpu.make_async_copy(
                bufs[u], out_hbm.at[pl.ds(b_loc * seq + s_base, S_WIN), :],
                ssems[u])

        def fill(b_loc, buf):
            xv = x_v[pl.ds(b_loc * S_WIN, S_WIN)]
            rb_vec = tb_off + (xv * S_WIN + iota) * dim
            for l in range(S_WIN):
                rb = jnp.reshape(lax.slice(rb_vec, (l,), (l + 1,)), ())
                rb = pl.multiple_of(rb, dim)
                pltpu.make_async_copy(
                    tbl_sp.at[pl.ds(rb, dim)], buf.at[l], fsem).start()
            pltpu.make_async_copy(
                out_hbm.at[pl.ds(0, S_WIN), :], buf, fsem).wait()

        for u in range(2):
            fill(u, bufs[u])
            scatter(u, u).start()

        def body(b2, _):
            for u in range(2):
                b_loc = b2 * 2 + u
                scatter(b_loc - 2, u).wait()
                fill(b_loc, bufs[u])
                scatter(b_loc, u).start()
            return 0

        lax.fori_loop(1, batch // 2, body, 0)
        scatter(batch - 2, 0).wait()
        scatter(batch - 1, 1).wait()

    return expand


@functools.partial(jax.jit, static_argnums=())
def kernel(x, token_table, pos_table, gamma, beta):
    batch, seq = x.shape
    vocab, dim = token_table.shape

    combined = pl.pallas_call(
        _combine_kernel,
        out_shape=jax.ShapeDtypeStruct((vocab, seq, dim), jnp.float32),
    )(token_table, pos_table[:seq], gamma, beta)

    x = x.astype(jnp.int32)
    combined_win = pl.pallas_call(
        _combine_win_kernel,
        out_shape=jax.ShapeDtypeStruct((seq // S_WIN, vocab, S_WIN, dim),
                                       jnp.float32),
    )(token_table, pos_table[:seq], gamma, beta)
    xw = jnp.transpose(x.reshape(batch, seq // S_WIN, S_WIN),
                       (1, 0, 2)).reshape(-1)
    out = _sc_expand_win(batch, seq, dim, vocab)(
        xw, combined_win.reshape(-1))
    return out.reshape(batch, seq, dim)
